# baseline (speedup 1.0000x reference)
import functools

import jax
import jax.numpy as jnp
from jax import lax
from jax.experimental import pallas as pl
from jax.experimental.pallas import tpu as pltpu
from jax.experimental.pallas import tpu_sc as plsc

N = 10000
E = 320000
D = 128
HD = D // 2

NC = 2
NS = 16
EPT = E // NS
CH = 80
NCH = 250
EPTP = NCH * CH
RPS = 624
REM = N - NS * RPS


@functools.lru_cache(maxsize=None)
def _build_sc_scatter():
    mesh = plsc.VectorSubcoreMesh(core_axis_name="c", subcore_axis_name="s")

    @functools.partial(
        pl.kernel,
        out_type=jax.ShapeDtypeStruct((NC, N, HD), jnp.float32),
        mesh=mesh,
        scratch_types=[
            pltpu.VMEM((NCH, CH), jnp.int32),
            pltpu.VMEM((NCH, CH), jnp.int32),
            pltpu.VMEM((NCH, CH), jnp.float32),
            pltpu.VMEM((CH, HD), jnp.float32),
            pltpu.VMEM((CH, HD), jnp.float32),
            pltpu.VMEM_SHARED((N, HD), jnp.float32),
            pltpu.SemaphoreType.DMA,
            pltpu.SemaphoreType.DMA,
            pltpu.SemaphoreType.DMA,
            pltpu.SemaphoreType.DMA,
        ],
        compiler_params=pltpu.CompilerParams(use_tc_tiling_on_sc=False),
    )
    def sc_scatter(m_hbm, src_hbm, dst_hbm, ew_hbm, z_hbm, out_hbm,
                   src_v, dst_v, ew_v, rows0_v, rows1_v, agg_s,
                   gsem0, gsem1, ssem0, ssem1):
        rows_b = (rows0_v, rows1_v)
        gsem = (gsem0, gsem1)
        ssem = (ssem0, ssem1)
        cid = lax.axis_index("c")
        sid = lax.axis_index("s")
        pltpu.sync_copy(z_hbm.at[pl.ds(sid * RPS, RPS)],
                        agg_s.at[pl.ds(sid * RPS, RPS)])

        @pl.when(sid == NS - 1)
        def _zero_rem():
            pltpu.sync_copy(z_hbm.at[pl.ds(NS * RPS, REM)],
                            agg_s.at[pl.ds(NS * RPS, REM)])

        pltpu.sync_copy(src_hbm.at[sid], src_v)
        pltpu.sync_copy(dst_hbm.at[sid], dst_v)
        pltpu.sync_copy(ew_hbm.at[sid], ew_v)
        plsc.subcore_barrier()

        def gather_copy(c, b):
            return pltpu.make_async_copy(m_hbm.at[cid].at[src_v.at[c]],
                                         rows_b[b], gsem[b])

        def scatter_copy(c, b):
            return pltpu.make_async_copy(rows_b[b], agg_s.at[dst_v.at[c]],
                                         ssem[b])

        def scale(c, b):
            for g in range(CH // 16):
                wv = ew_v[c, pl.ds(g * 16, 16)]
                for j in range(16):
                    w = wv[j]
                    e = g * 16 + j
                    for dd in range(HD // 16):
                        sl = pl.ds(dd * 16, 16)
                        rows_b[b][e, sl] = rows_b[b][e, sl] * w

        gather_copy(0, 0).start()

        def pair(i, carry):
            for b in range(2):
                c = 2 * i + b
                nb = 1 - b

                @pl.when(c >= 1)
                def _drain_prev_scatter():
                    scatter_copy(c - 1, nb).wait()

                @pl.when(c + 1 < NCH)
                def _prefetch_next():
                    gather_copy(c + 1, nb).start()

                gather_copy(c, b).wait()
                scale(c, b)
                scatter_copy(c, b).start(add=True)
            return carry

        lax.fori_loop(0, NCH // 2, pair, 0)
        scatter_copy(NCH - 1, 1).wait()
        plsc.subcore_barrier()
        pltpu.sync_copy(agg_s.at[pl.ds(sid * RPS, RPS)],
                        out_hbm.at[cid, pl.ds(sid * RPS, RPS)])

        @pl.when(sid == NS - 1)
        def _out_rem():
            pltpu.sync_copy(agg_s.at[pl.ds(NS * RPS, REM)],
                            out_hbm.at[cid, pl.ds(NS * RPS, REM)])

    return sc_scatter


def _sc_scatter(m2, src, dst, ew, zeros):
    return _build_sc_scatter()(m2, src, dst, ew, zeros)


def _dot(a, b):
    return jnp.dot(a, b, preferred_element_type=jnp.float32,
                   precision=lax.Precision.HIGHEST)


def _split_cols(m, o_ref):
    o_ref[0] = m[:, :HD]
    o_ref[1] = m[:, HD:]


def _mm_body(h_ref, w_ref, o_ref):
    _split_cols(_dot(h_ref[...], w_ref[...]), o_ref)


def _msplit(bm):
    return pl.BlockSpec((NC, bm, HD), lambda i: (0, i, 0))


def _rows(bm):
    return pl.BlockSpec((bm, D), lambda i: (i, 0))


def _rep(shape):
    nd = len(shape)
    return pl.BlockSpec(shape, lambda i, _n=nd: (0,) * _n)


def _matmul_split(h, w, bm=1000):
    return pl.pallas_call(
        _mm_body,
        grid=(N // bm,),
        in_specs=[_rows(bm), _rep((D, D))],
        out_specs=_msplit(bm),
        out_shape=jax.ShapeDtypeStruct((NC, N, HD), jnp.float32),
    )(h, w)


def _gru(p_ref, h_ref, wih_ref, whh_ref, bih_ref, bhh_ref, h_out):
    agg = jnp.concatenate([p_ref[0], p_ref[1]], axis=1)
    h = h_ref[...]
    gi = _dot(agg, wih_ref[...]) + bih_ref[...]
    gh = _dot(h, whh_ref[...]) + bhh_ref[...]
    r = jax.nn.sigmoid(gi[:, :D] + gh[:, :D])
    z = jax.nn.sigmoid(gi[:, D:2 * D] + gh[:, D:2 * D])
    n = jnp.tanh(gi[:, 2 * D:] + r * gh[:, 2 * D:])
    h_out[...] = (1.0 - z) * n + z * h


def _gru_next_body(p_ref, h_ref, wih_ref, whh_ref, bih_ref, bhh_ref,
                   wn_ref, h_out, m_out):
    _gru(p_ref, h_ref, wih_ref, whh_ref, bih_ref, bhh_ref, h_out)
    _split_cols(_dot(h_out[...], wn_ref[...]), m_out)


def _gru_lstm_body(p_ref, h_ref, wih_ref, whh_ref, bih_ref, bhh_ref,
                   h0_ref, c0_ref, lwih_ref, lwhh_ref, lb_ref,
                   linw_ref, linb_ref, out_ref, h1_ref, c1_ref):
    _gru(p_ref, h_ref, wih_ref, whh_ref, bih_ref, bhh_ref, h1_ref)
    hc = h1_ref[...]
    gates = (_dot(hc, lwih_ref[...]) + _dot(h0_ref[...], lwhh_ref[...])
             + lb_ref[...])
    L = D
    ig = jax.nn.sigmoid(gates[:, :L])
    fg = jax.nn.sigmoid(gates[:, L:2 * L])
    gg = jnp.tanh(gates[:, 2 * L:3 * L])
    og = jax.nn.sigmoid(gates[:, 3 * L:])
    c1 = fg * c0_ref[...] + ig * gg
    h1 = og * jnp.tanh(c1)
    c1_ref[...] = c1
    h1_ref[...] = h1
    out_ref[...] = _dot(jnp.maximum(h1, 0.0), linw_ref[...]) + linb_ref[...]


def _gru_next(p, h, wihT, whhT, bih, bhh, wnext, bm=1000):
    specs = [_msplit(bm), _rows(bm),
             _rep((D, 3 * D)), _rep((D, 3 * D)),
             _rep((1, 3 * D)), _rep((1, 3 * D)), _rep((D, D))]
    return pl.pallas_call(
        _gru_next_body,
        grid=(N // bm,),
        in_specs=specs,
        out_specs=[_rows(bm), _msplit(bm)],
        out_shape=[jax.ShapeDtypeStruct((N, D), jnp.float32),
                   jax.ShapeDtypeStruct((NC, N, HD), jnp.float32)],
    )(p, h, wihT, whhT, bih, bhh, wnext)


def _gru_lstm(p, h, wihT, whhT, bih, bhh, h0, c0, lwihT, lwhhT, lb,
              linwT, linb, bm=1000):
    specs = [_msplit(bm), _rows(bm),
             _rep((D, 3 * D)), _rep((D, 3 * D)),
             _rep((1, 3 * D)), _rep((1, 3 * D)),
             _rows(bm), _rows(bm),
             _rep((D, 4 * D)), _rep((D, 4 * D)), _rep((1, 4 * D)),
             _rep((D, D)), _rep((1, D))]
    hs = jax.ShapeDtypeStruct((N, D), jnp.float32)
    return pl.pallas_call(
        _gru_lstm_body,
        grid=(N // bm,),
        in_specs=specs,
        out_specs=[_rows(bm), _rows(bm), _rows(bm)],
        out_shape=[hs, hs, hs],
    )(p, h, wihT, whhT, bih, bhh, h0, c0, lwihT, lwhhT, lb, linwT, linb)


def kernel(x, edge_index, edge_weight, H, C, ggc_weight, gru_w_ih, gru_w_hh,
           gru_b_ih, gru_b_hh, lstm_w_ih, lstm_w_hh, lstm_b_ih, lstm_b_hh,
           lin_w, lin_b):
    pad = ((0, 0), (0, EPTP - EPT))
    src = jnp.pad(edge_index[0].reshape(NS, EPT), pad).reshape(NS, NCH, CH)
    spread = jnp.broadcast_to(jnp.arange(EPTP - EPT, dtype=jnp.int32) * 31
                              % N, (NS, EPTP - EPT))
    dst = jnp.concatenate([edge_index[1].reshape(NS, EPT), spread],
                          axis=1).reshape(NS, NCH, CH)
    ew = jnp.pad(edge_weight.reshape(NS, EPT), pad).reshape(NS, NCH, CH)
    zeros = jnp.zeros((N, HD), jnp.float32)

    wihT = gru_w_ih.T
    whhT = gru_w_hh.T
    bih = gru_b_ih.reshape(1, 3 * D)
    bhh = gru_b_hh.reshape(1, 3 * D)
    lwihT = lstm_w_ih.T
    lwhhT = lstm_w_hh.T
    lb = (lstm_b_ih + lstm_b_hh).reshape(1, 4 * D)
    linwT = lin_w.T
    linb = lin_b.reshape(1, D)

    m1 = _matmul_split(x, ggc_weight[0])
    p1 = _sc_scatter(m1, src, dst, ew, zeros)
    h1c, m2 = _gru_next(p1, x, wihT, whhT, bih, bhh, ggc_weight[1])
    p2 = _sc_scatter(m2, src, dst, ew, zeros)
    out, h1, c1 = _gru_lstm(p2, h1c, wihT, whhT, bih, bhh, H[0], C[0],
                            lwihT, lwhhT, lb, linwT, linb)
    return out, h1[None, :, :], c1[None, :, :]


# speedup vs baseline: 1.2439x; 1.2439x over previous
import functools

import jax
import jax.numpy as jnp
from jax import lax
from jax.experimental import pallas as pl
from jax.experimental.pallas import tpu as pltpu
from jax.experimental.pallas import tpu_sc as plsc

N = 10000
E = 320000
D = 128
HD = D // 2

NC = 2
NS = 16
EPT = E // NS
CH = 80
NCH = 250
EPTP = NCH * CH
RPS = 624
REM = N - NS * RPS


@functools.lru_cache(maxsize=None)
def _build_sc_scatter():
    mesh = plsc.VectorSubcoreMesh(core_axis_name="c", subcore_axis_name="s")

    @functools.partial(
        pl.kernel,
        out_type=jax.ShapeDtypeStruct((NC, N, HD), jnp.float32),
        mesh=mesh,
        scratch_types=[
            pltpu.VMEM((NCH, CH), jnp.int32),
            pltpu.VMEM((NCH, CH), jnp.int32),
            pltpu.VMEM((NCH, CH), jnp.float32),
            pltpu.VMEM((CH, HD), jnp.float32),
            pltpu.VMEM((CH, HD), jnp.float32),
            pltpu.VMEM_SHARED((N, HD), jnp.float32),
            pltpu.SemaphoreType.DMA,
            pltpu.SemaphoreType.DMA,
            pltpu.SemaphoreType.DMA,
            pltpu.SemaphoreType.DMA,
        ],
        compiler_params=pltpu.CompilerParams(use_tc_tiling_on_sc=False),
    )
    def sc_scatter(m_hbm, src_hbm, dst_hbm, ew_hbm, z_hbm, out_hbm,
                   src_v, dst_v, ew_v, rows0_v, rows1_v, agg_s,
                   gsem0, gsem1, ssem0, ssem1):
        rows_b = (rows0_v, rows1_v)
        gsem = (gsem0, gsem1)
        ssem = (ssem0, ssem1)
        cid = lax.axis_index("c")
        sid = lax.axis_index("s")
        pltpu.sync_copy(z_hbm.at[pl.ds(sid * RPS, RPS)],
                        agg_s.at[pl.ds(sid * RPS, RPS)])

        @pl.when(sid == NS - 1)
        def _zero_rem():
            pltpu.sync_copy(z_hbm.at[pl.ds(NS * RPS, REM)],
                            agg_s.at[pl.ds(NS * RPS, REM)])

        pltpu.sync_copy(src_hbm.at[sid], src_v)
        pltpu.sync_copy(dst_hbm.at[sid], dst_v)
        pltpu.sync_copy(ew_hbm.at[sid], ew_v)
        plsc.subcore_barrier()

        def gather_copy(c, b):
            return pltpu.make_async_copy(m_hbm.at[cid].at[src_v.at[c]],
                                         rows_b[b], gsem[b])

        def scatter_copy(c, b):
            return pltpu.make_async_copy(rows_b[b], agg_s.at[dst_v.at[c]],
                                         ssem[b])

        def scale(c, b):
            for g in range(CH // 16):
                wv = ew_v[c, pl.ds(g * 16, 16)]
                for j in range(16):
                    w = wv[j]
                    e = g * 16 + j
                    for dd in range(HD // 16):
                        sl = pl.ds(dd * 16, 16)
                        rows_b[b][e, sl] = rows_b[b][e, sl] * w

        gather_copy(0, 0).start()

        def pair(i, carry):
            for b in range(2):
                c = 2 * i + b
                nb = 1 - b

                @pl.when(c >= 1)
                def _drain_prev_scatter():
                    scatter_copy(c - 1, nb).wait()

                @pl.when(c + 1 < NCH)
                def _prefetch_next():
                    gather_copy(c + 1, nb).start()

                gather_copy(c, b).wait()
                scale(c, b)
                scatter_copy(c, b).start(add=True)
            return carry

        lax.fori_loop(0, NCH // 2, pair, 0)
        scatter_copy(NCH - 1, 1).wait()
        plsc.subcore_barrier()
        pltpu.sync_copy(agg_s.at[pl.ds(sid * RPS, RPS)],
                        out_hbm.at[cid, pl.ds(sid * RPS, RPS)])

        @pl.when(sid == NS - 1)
        def _out_rem():
            pltpu.sync_copy(agg_s.at[pl.ds(NS * RPS, REM)],
                            out_hbm.at[cid, pl.ds(NS * RPS, REM)])

    return sc_scatter


def _sc_scatter(m2, src, dst, ew, zeros):
    return _build_sc_scatter()(m2, src, dst, ew, zeros)


def _dot(a, b):
    return jnp.dot(a, b, preferred_element_type=jnp.float32,
                   precision=lax.Precision.DEFAULT)


def _split_cols(m, o_ref):
    o_ref[0] = m[:, :HD]
    o_ref[1] = m[:, HD:]


def _mm_body(h_ref, w_ref, o_ref):
    _split_cols(_dot(h_ref[...], w_ref[...]), o_ref)


def _msplit(bm):
    return pl.BlockSpec((NC, bm, HD), lambda i: (0, i, 0))


def _rows(bm):
    return pl.BlockSpec((bm, D), lambda i: (i, 0))


def _rep(shape):
    nd = len(shape)
    return pl.BlockSpec(shape, lambda i, _n=nd: (0,) * _n)


def _matmul_split(h, w, bm=1000):
    return pl.pallas_call(
        _mm_body,
        grid=(N // bm,),
        in_specs=[_rows(bm), _rep((D, D))],
        out_specs=_msplit(bm),
        out_shape=jax.ShapeDtypeStruct((NC, N, HD), jnp.float32),
    )(h, w)


def _gru(p_ref, h_ref, wih_ref, whh_ref, bih_ref, bhh_ref, h_out):
    agg = jnp.concatenate([p_ref[0], p_ref[1]], axis=1)
    h = h_ref[...]
    gi = _dot(agg, wih_ref[...]) + bih_ref[...]
    gh = _dot(h, whh_ref[...]) + bhh_ref[...]
    r = jax.nn.sigmoid(gi[:, :D] + gh[:, :D])
    z = jax.nn.sigmoid(gi[:, D:2 * D] + gh[:, D:2 * D])
    n = jnp.tanh(gi[:, 2 * D:] + r * gh[:, 2 * D:])
    h_out[...] = (1.0 - z) * n + z * h


def _gru_next_body(p_ref, h_ref, wih_ref, whh_ref, bih_ref, bhh_ref,
                   wn_ref, h_out, m_out):
    _gru(p_ref, h_ref, wih_ref, whh_ref, bih_ref, bhh_ref, h_out)
    _split_cols(_dot(h_out[...], wn_ref[...]), m_out)


def _gru_lstm_body(p_ref, h_ref, wih_ref, whh_ref, bih_ref, bhh_ref,
                   h0_ref, c0_ref, lwih_ref, lwhh_ref, lb_ref,
                   linw_ref, linb_ref, out_ref, h1_ref, c1_ref):
    _gru(p_ref, h_ref, wih_ref, whh_ref, bih_ref, bhh_ref, h1_ref)
    hc = h1_ref[...]
    gates = (_dot(hc, lwih_ref[...]) + _dot(h0_ref[...], lwhh_ref[...])
             + lb_ref[...])
    L = D
    ig = jax.nn.sigmoid(gates[:, :L])
    fg = jax.nn.sigmoid(gates[:, L:2 * L])
    gg = jnp.tanh(gates[:, 2 * L:3 * L])
    og = jax.nn.sigmoid(gates[:, 3 * L:])
    c1 = fg * c0_ref[...] + ig * gg
    h1 = og * jnp.tanh(c1)
    c1_ref[...] = c1
    h1_ref[...] = h1
    out_ref[...] = _dot(jnp.maximum(h1, 0.0), linw_ref[...]) + linb_ref[...]


def _gru_next(p, h, wihT, whhT, bih, bhh, wnext, bm=1000):
    specs = [_msplit(bm), _rows(bm),
             _rep((D, 3 * D)), _rep((D, 3 * D)),
             _rep((1, 3 * D)), _rep((1, 3 * D)), _rep((D, D))]
    return pl.pallas_call(
        _gru_next_body,
        grid=(N // bm,),
        in_specs=specs,
        out_specs=[_rows(bm), _msplit(bm)],
        out_shape=[jax.ShapeDtypeStruct((N, D), jnp.float32),
                   jax.ShapeDtypeStruct((NC, N, HD), jnp.float32)],
    )(p, h, wihT, whhT, bih, bhh, wnext)


def _gru_lstm(p, h, wihT, whhT, bih, bhh, h0, c0, lwihT, lwhhT, lb,
              linwT, linb, bm=1000):
    specs = [_msplit(bm), _rows(bm),
             _rep((D, 3 * D)), _rep((D, 3 * D)),
             _rep((1, 3 * D)), _rep((1, 3 * D)),
             _rows(bm), _rows(bm),
             _rep((D, 4 * D)), _rep((D, 4 * D)), _rep((1, 4 * D)),
             _rep((D, D)), _rep((1, D))]
    hs = jax.ShapeDtypeStruct((N, D), jnp.float32)
    return pl.pallas_call(
        _gru_lstm_body,
        grid=(N // bm,),
        in_specs=specs,
        out_specs=[_rows(bm), _rows(bm), _rows(bm)],
        out_shape=[hs, hs, hs],
    )(p, h, wihT, whhT, bih, bhh, h0, c0, lwihT, lwhhT, lb, linwT, linb)


def kernel(x, edge_index, edge_weight, H, C, ggc_weight, gru_w_ih, gru_w_hh,
           gru_b_ih, gru_b_hh, lstm_w_ih, lstm_w_hh, lstm_b_ih, lstm_b_hh,
           lin_w, lin_b):
    pad = ((0, 0), (0, EPTP - EPT))
    src = jnp.pad(edge_index[0].reshape(NS, EPT), pad).reshape(NS, NCH, CH)
    spread = jnp.broadcast_to(jnp.arange(EPTP - EPT, dtype=jnp.int32) * 31
                              % N, (NS, EPTP - EPT))
    dst = jnp.concatenate([edge_index[1].reshape(NS, EPT), spread],
                          axis=1).reshape(NS, NCH, CH)
    ew = jnp.pad(edge_weight.reshape(NS, EPT), pad).reshape(NS, NCH, CH)
    zeros = jnp.zeros((N, HD), jnp.float32)

    wihT = gru_w_ih.T
    whhT = gru_w_hh.T
    bih = gru_b_ih.reshape(1, 3 * D)
    bhh = gru_b_hh.reshape(1, 3 * D)
    lwihT = lstm_w_ih.T
    lwhhT = lstm_w_hh.T
    lb = (lstm_b_ih + lstm_b_hh).reshape(1, 4 * D)
    linwT = lin_w.T
    linb = lin_b.reshape(1, D)

    m1 = _matmul_split(x, ggc_weight[0])
    p1 = _sc_scatter(m1, src, dst, ew, zeros)
    h1c, m2 = _gru_next(p1, x, wihT, whhT, bih, bhh, ggc_weight[1])
    p2 = _sc_scatter(m2, src, dst, ew, zeros)
    out, h1, c1 = _gru_lstm(p2, h1c, wihT, whhT, bih, bhh, H[0], C[0],
                            lwihT, lwhhT, lb, linwT, linb)
    return out, h1[None, :, :], c1[None, :, :]
